# baseline (speedup 1.0000x reference)
import functools

import jax
import jax.numpy as jnp
from jax import lax
from jax.experimental import pallas as pl
from jax.experimental.pallas import tpu as pltpu
from jax.experimental.pallas import tpu_sc as plsc

NC = 2
NS = 16
NW = NC * NS
DCH = 128
NDB = 4
CH = 64
NBUF = 4
UR = 8
BLK = 1280


def _make_sc_kernel(n_pad, e_pad, d):
  per_w = e_pad // NW
  n_chunks = per_w // CH
  n_rounds = n_chunks // NBUF
  dchunks = e_pad // NS // DCH
  rows_per_s = n_pad // NS

  mesh = plsc.VectorSubcoreMesh(
      core_axis_name="c", subcore_axis_name="s",
      num_cores=NC, num_subcores=NS)

  @functools.partial(
      pl.kernel,
      out_type=(jax.ShapeDtypeStruct((n_pad, d), jnp.float32),
                jax.ShapeDtypeStruct((n_pad, d), jnp.float32),
                jax.ShapeDtypeStruct((n_pad, d), jnp.float32),
                jax.ShapeDtypeStruct((n_pad,), jnp.float32)),
      mesh=mesh,
      scratch_types=[
          pltpu.VMEM_SHARED((n_pad, d), jnp.float32),
          pltpu.VMEM_SHARED((n_pad,), jnp.float32),
          pltpu.VMEM((per_w,), jnp.int32),
          pltpu.VMEM((rows_per_s,), jnp.float32),
          pltpu.VMEM((rows_per_s,), jnp.float32),
          pltpu.VMEM((DCH,), jnp.float32),
      ] + [pltpu.VMEM((DCH,), jnp.int32)] * NDB
        + [pltpu.VMEM((CH,), jnp.int32)] * NBUF
        + [pltpu.VMEM((CH, d), jnp.float32)] * NBUF
        + [pltpu.SemaphoreType.DMA] * (2 * NDB + 3 * NBUF + 1),
  )
  def sc_kernel(x_hbm, src_hbm, dst_hbm, dstd_hbm, zeros_hbm,
                p0_hbm, p1_hbm, x2_hbm, deg_hbm,
                acc_sh, deg_sh, srcs_v, degb, disb, ones_v,
                *bufs_and_sems):
    didx = bufs_and_sems[:NDB]
    dstb = bufs_and_sems[NDB:NDB + NBUF]
    rows = bufs_and_sems[NDB + NBUF:NDB + 2 * NBUF]
    sems = bufs_and_sems[NDB + 2 * NBUF:]
    dgs = sems[:NDB]
    dsc = sems[NDB:2 * NDB]
    gsem = sems[2 * NDB:2 * NDB + NBUF]
    ssem = sems[2 * NDB + NBUF:2 * NDB + 2 * NBUF]
    dsem = sems[2 * NDB + 2 * NBUF:2 * NDB + 3 * NBUF]
    isem = sems[2 * NDB + 3 * NBUF]

    c = lax.axis_index("c")
    sax = lax.axis_index("s")
    wid = c * NS + sax
    row0 = sax * rows_per_s

    for j in range(DCH // 16):
      ones_v[pl.ds(j * 16, 16)] = jnp.ones((16,), jnp.float32)
    pltpu.sync_copy(zeros_hbm.at[pl.ds(row0, rows_per_s)],
                    deg_sh.at[pl.ds(row0, rows_per_s)])
    pltpu.async_copy(src_hbm.at[wid], srcs_v, isem)
    plsc.subcore_barrier()

    dbase = sax * dchunks
    for t in range(NDB):
      pltpu.async_copy(dstd_hbm.at[dbase + t], didx[t], dgs[t])

    def deg_round(j, carry):
      for t in range(NDB):
        pltpu.make_async_copy(dstd_hbm.at[0], didx[t], dgs[t]).wait()
        pltpu.async_copy(ones_v, deg_sh.at[didx[t]], dsc[t], add=True)
      for t in range(NDB):
        nxt = NDB * (j + 1) + t
        pltpu.make_async_copy(zeros_hbm.at[pl.ds(0, DCH)], ones_v,
                              dsc[t]).wait()

        @pl.when(nxt < dchunks)
        def _():
          pltpu.async_copy(dstd_hbm.at[dbase + nxt], didx[t], dgs[t])

      return carry

    lax.fori_loop(0, dchunks // NDB, deg_round, 0)
    plsc.subcore_barrier()

    @pl.when(c == 0)
    def _():
      pltpu.sync_copy(deg_sh.at[pl.ds(row0, rows_per_s)],
                      deg_hbm.at[pl.ds(row0, rows_per_s)])

    pltpu.sync_copy(deg_sh.at[pl.ds(row0, rows_per_s)], degb)

    def dis_body(k, carry):
      v = degb[pl.ds(k * 16, 16)] + 1.0
      bits = lax.bitcast_convert_type(v, jnp.int32)
      y = lax.bitcast_convert_type(
          jnp.int32(0x5F3759DF) - lax.shift_right_logical(bits, 1),
          jnp.float32)
      for _ in range(3):
        y = y * (1.5 - 0.5 * v * y * y)
      disb[pl.ds(k * 16, 16)] = y
      return carry

    lax.fori_loop(0, rows_per_s // 16, dis_body, 0)

    n_xch = rows_per_s // CH

    def x_body(q, carry):
      r0 = row0 + q * CH
      pltpu.sync_copy(x_hbm.at[pl.ds(r0, CH)], rows[0])

      def r_body(rr, carry2):
        dv16 = disb[pl.ds(q * CH + rr * 16, 16)]
        for u in range(16):
          r = rr * 16 + u
          dv = lax.gather(
              dv16, jnp.full((16, 1), u, jnp.int32),
              lax.GatherDimensionNumbers(offset_dims=(),
                                         collapsed_slice_dims=(0,),
                                         start_index_map=(0,)),
              (1,), mode=lax.GatherScatterMode.PROMISE_IN_BOUNDS)
          for k in range(d // 16):
            rows[0][r, pl.ds(k * 16, 16)] = (
                rows[0][r, pl.ds(k * 16, 16)] * dv)
        return carry2

      lax.fori_loop(0, CH // 16, r_body, 0)
      pltpu.sync_copy(rows[0], acc_sh.at[pl.ds(r0, CH)])
      pltpu.sync_copy(rows[0], x2_hbm.at[pl.ds(r0, CH)])
      return carry

    lax.fori_loop(0, n_xch, x_body, 0)
    pltpu.make_async_copy(src_hbm.at[0], srcs_v, isem).wait()
    plsc.subcore_barrier()

    dst_base = wid * n_chunks
    for b in range(NBUF):
      pltpu.async_copy(dst_hbm.at[dst_base + b], dstb[b], dsem[b])
      pltpu.async_copy(x2_hbm.at[srcs_v.at[pl.ds(b * CH, CH)]],
                       rows[b], gsem[b])

    def edge_round(j, carry):
      for b in range(NBUF):
        pltpu.make_async_copy(dst_hbm.at[0], dstb[b], dsem[b]).wait()
        pltpu.make_async_copy(x2_hbm.at[pl.ds(0, CH)], rows[b],
                              gsem[b]).wait()
        pltpu.async_copy(rows[b], acc_sh.at[dstb[b]], ssem[b], add=True)
      for b in range(NBUF):
        nxt = NBUF * (j + 1) + b
        pltpu.make_async_copy(x2_hbm.at[pl.ds(0, CH)], rows[b],
                              ssem[b]).wait()

        @pl.when(nxt < n_chunks)
        def _():
          pltpu.async_copy(dst_hbm.at[dst_base + nxt], dstb[b], dsem[b])
          pltpu.async_copy(x2_hbm.at[srcs_v.at[pl.ds(nxt * CH, CH)]],
                           rows[b], gsem[b])

      return carry

    lax.fori_loop(0, n_rounds, edge_round, 0)
    plsc.subcore_barrier()

    @pl.when(c == 0)
    def _():
      pltpu.sync_copy(acc_sh.at[pl.ds(row0, rows_per_s)],
                      p0_hbm.at[pl.ds(row0, rows_per_s)])

    @pl.when(c == 1)
    def _():
      pltpu.sync_copy(acc_sh.at[pl.ds(row0, rows_per_s)],
                      p1_hbm.at[pl.ds(row0, rows_per_s)])

  return sc_kernel


def _finalize_body(p0_ref, p1_ref, x_ref, deg_ref, w_ref,
                   b_ref, g_ref, be_ref, o_ref):
  dis = lax.rsqrt(1.0 + deg_ref[...])
  aggr = p0_ref[...] + p1_ref[...] - x_ref[...] * dis
  out = jnp.dot(aggr, w_ref[...],
                preferred_element_type=jnp.float32) * dis + b_ref[...]
  mu = jnp.mean(out, axis=-1, keepdims=True)
  var = jnp.mean((out - mu) ** 2, axis=-1, keepdims=True)
  ln = (out - mu) * lax.rsqrt(var + 1e-5) * g_ref[...] + be_ref[...]
  o_ref[...] = jnp.maximum(ln + x_ref[...], 0.0)


def kernel(x, edge_index, W, b, gamma, beta):
  n, d = x.shape
  e = edge_index.shape[1]
  n_pad = ((n + BLK) // BLK) * BLK
  egrain = NW * max(CH * NBUF, DCH)
  e_pad = ((e + egrain - 1) // egrain) * egrain
  per_w = e_pad // NW
  n_chunks = per_w // CH
  dchunks = e_pad // NS // DCH

  src = edge_index[0].astype(jnp.int32)
  dst = edge_index[1].astype(jnp.int32)
  pad_idx = n + (jnp.arange(e_pad - e, dtype=jnp.int32) % (n_pad - n))
  src2 = jnp.concatenate([src, pad_idx]).reshape(NW, per_w)
  dst_flat = jnp.concatenate([dst, pad_idx])
  dst2 = dst_flat.reshape(NW * n_chunks, CH)
  dstd = dst_flat.reshape(NS * dchunks, DCH)
  x_pad = jnp.concatenate([x, jnp.zeros((n_pad - n, d), jnp.float32)])

  p0, p1, _, deg = _make_sc_kernel(n_pad, e_pad, d)(
      x_pad, src2, dst2, dstd, jnp.zeros((n_pad,), jnp.float32))

  grid = (n_pad // BLK,)
  row_spec = pl.BlockSpec((BLK, d), lambda i: (i, 0))
  col_spec = pl.BlockSpec((BLK, 1), lambda i: (i, 0))
  vec_spec = pl.BlockSpec((1, d), lambda i: (0, 0))

  out_pad = pl.pallas_call(
      _finalize_body,
      grid=grid,
      in_specs=[row_spec, row_spec, row_spec, col_spec,
                pl.BlockSpec((d, d), lambda i: (0, 0)),
                vec_spec, vec_spec, vec_spec],
      out_specs=row_spec,
      out_shape=jax.ShapeDtypeStruct((n_pad, d), jnp.float32),
  )(p0, p1, x_pad, deg.reshape(n_pad, 1), W,
    b.reshape(1, d), gamma.reshape(1, d), beta.reshape(1, d))

  return out_pad[:n]


# speedup vs baseline: 1.1154x; 1.1154x over previous
import functools

import jax
import jax.numpy as jnp
from jax import lax
from jax.experimental import pallas as pl
from jax.experimental.pallas import tpu as pltpu
from jax.experimental.pallas import tpu_sc as plsc

NC = 2
NS = 16
NW = NC * NS
DCH = 128
CH = 64
NBUF = 4
BLK = 1280


def _sc_mesh():
  return plsc.VectorSubcoreMesh(
      core_axis_name="c", subcore_axis_name="s",
      num_cores=NC, num_subcores=NS)


def _make_deg_kernel(n_pad, e_pad):
  per_w = e_pad // NW
  n_chunks = per_w // DCH
  rows_per_s = n_pad // NS

  @functools.partial(
      pl.kernel,
      out_type=(jax.ShapeDtypeStruct((n_pad,), jnp.float32),
                jax.ShapeDtypeStruct((n_pad,), jnp.float32)),
      mesh=_sc_mesh(),
      scratch_types=[
          pltpu.VMEM_SHARED((n_pad,), jnp.float32),
          pltpu.VMEM((n_chunks, DCH), jnp.int32),
          pltpu.VMEM((DCH,), jnp.float32),
          pltpu.SemaphoreType.DMA,
      ],
  )
  def deg_kernel(dst_hbm, zeros_hbm, d0_hbm, d1_hbm,
                 deg_sh, dsts_v, ones_v, sem):
    c = lax.axis_index("c")
    sax = lax.axis_index("s")
    wid = c * NS + sax
    for j in range(DCH // 16):
      ones_v[pl.ds(j * 16, 16)] = jnp.ones((16,), jnp.float32)
    row0 = sax * rows_per_s
    pltpu.sync_copy(zeros_hbm.at[pl.ds(row0, rows_per_s)],
                    deg_sh.at[pl.ds(row0, rows_per_s)])
    pltpu.sync_copy(dst_hbm.at[wid], dsts_v)
    plsc.subcore_barrier()

    def fire(i, carry):
      pltpu.async_copy(ones_v, deg_sh.at[dsts_v.at[i]], sem, add=True)
      return carry

    lax.fori_loop(0, n_chunks, fire, 0)

    def drain(i, carry):
      pltpu.make_async_copy(zeros_hbm.at[pl.ds(0, DCH)], ones_v, sem).wait()
      return carry

    lax.fori_loop(0, n_chunks, drain, 0)
    plsc.subcore_barrier()

    @pl.when(c == 0)
    def _():
      pltpu.sync_copy(deg_sh.at[pl.ds(row0, rows_per_s)],
                      d0_hbm.at[pl.ds(row0, rows_per_s)])

    @pl.when(c == 1)
    def _():
      pltpu.sync_copy(deg_sh.at[pl.ds(row0, rows_per_s)],
                      d1_hbm.at[pl.ds(row0, rows_per_s)])

  return deg_kernel


def _make_edge_kernel(n_pad, e_pad, d):
  per_w = e_pad // NW
  n_chunks = per_w // CH
  n_rounds = n_chunks // NBUF
  rows_per_s = n_pad // NS

  @functools.partial(
      pl.kernel,
      out_type=(jax.ShapeDtypeStruct((n_pad, d), jnp.float32),
                jax.ShapeDtypeStruct((n_pad, d), jnp.float32)),
      mesh=_sc_mesh(),
      scratch_types=[
          pltpu.VMEM_SHARED((n_pad, d), jnp.float32),
          pltpu.VMEM((per_w,), jnp.int32),
      ] + [pltpu.VMEM((CH,), jnp.int32)] * NBUF
        + [pltpu.VMEM((CH, d), jnp.float32)] * NBUF
        + [pltpu.SemaphoreType.DMA] * (3 * NBUF),
  )
  def edge_kernel(h2_hbm, src_hbm, dst_hbm, p0_hbm, p1_hbm,
                  acc_sh, srcs_v, *bufs_and_sems):
    dstb = bufs_and_sems[:NBUF]
    rows = bufs_and_sems[NBUF:2 * NBUF]
    gsem = bufs_and_sems[2 * NBUF:3 * NBUF]
    ssem = bufs_and_sems[3 * NBUF:4 * NBUF]
    dsem = bufs_and_sems[4 * NBUF:]
    c = lax.axis_index("c")
    sax = lax.axis_index("s")
    wid = c * NS + sax
    row0 = sax * rows_per_s
    init_cp = pltpu.async_copy(h2_hbm.at[pl.ds(row0, rows_per_s)],
                               acc_sh.at[pl.ds(row0, rows_per_s)], gsem[0])
    src_cp = pltpu.async_copy(src_hbm.at[wid], srcs_v, gsem[1])
    init_cp.wait()
    src_cp.wait()
    plsc.subcore_barrier()

    dst_base = wid * n_chunks
    for b in range(NBUF):
      pltpu.async_copy(dst_hbm.at[dst_base + b], dstb[b], dsem[b])
      pltpu.async_copy(h2_hbm.at[srcs_v.at[pl.ds(b * CH, CH)]],
                       rows[b], gsem[b])

    def round_body(j, carry):
      for b in range(NBUF):
        pltpu.make_async_copy(dst_hbm.at[0], dstb[b], dsem[b]).wait()
        pltpu.make_async_copy(h2_hbm.at[pl.ds(0, CH)], rows[b],
                              gsem[b]).wait()
        pltpu.async_copy(rows[b], acc_sh.at[dstb[b]], ssem[b], add=True)
      for b in range(NBUF):
        nxt = NBUF * (j + 1) + b
        pltpu.make_async_copy(h2_hbm.at[pl.ds(0, CH)], rows[b],
                              ssem[b]).wait()

        @pl.when(nxt < n_chunks)
        def _():
          pltpu.async_copy(dst_hbm.at[dst_base + nxt], dstb[b], dsem[b])
          pltpu.async_copy(h2_hbm.at[srcs_v.at[pl.ds(nxt * CH, CH)]],
                           rows[b], gsem[b])

      return carry

    lax.fori_loop(0, n_rounds, round_body, 0)
    plsc.subcore_barrier()

    @pl.when(c == 0)
    def _():
      pltpu.sync_copy(acc_sh.at[pl.ds(row0, rows_per_s)],
                      p0_hbm.at[pl.ds(row0, rows_per_s)])

    @pl.when(c == 1)
    def _():
      pltpu.sync_copy(acc_sh.at[pl.ds(row0, rows_per_s)],
                      p1_hbm.at[pl.ds(row0, rows_per_s)])

  return edge_kernel


def _prescale_body(x_ref, w_ref, d0_ref, d1_ref, h2_ref):
  dis = lax.rsqrt(1.0 + d0_ref[...] + d1_ref[...])
  h2_ref[...] = jnp.dot(x_ref[...] * dis, w_ref[...],
                        preferred_element_type=jnp.float32)


def _finalize_body(p0_ref, p1_ref, h2_ref, x_ref, d0_ref, d1_ref,
                   b_ref, g_ref, be_ref, o_ref):
  dis = lax.rsqrt(1.0 + d0_ref[...] + d1_ref[...])
  acc = p0_ref[...] + p1_ref[...] - h2_ref[...]
  out = acc * dis + b_ref[...]
  mu = jnp.mean(out, axis=-1, keepdims=True)
  var = jnp.mean((out - mu) ** 2, axis=-1, keepdims=True)
  ln = (out - mu) * lax.rsqrt(var + 1e-5) * g_ref[...] + be_ref[...]
  o_ref[...] = jnp.maximum(ln + x_ref[...], 0.0)


def kernel(x, edge_index, W, b, gamma, beta):
  n, d = x.shape
  e = edge_index.shape[1]
  n_pad = ((n + BLK) // BLK) * BLK
  egrain = NW * max(CH * NBUF, DCH)
  e_pad = ((e + egrain - 1) // egrain) * egrain
  per_w = e_pad // NW
  n_chunks = per_w // CH

  src = edge_index[0].astype(jnp.int32)
  dst = edge_index[1].astype(jnp.int32)
  pad_idx = n + (jnp.arange(e_pad - e, dtype=jnp.int32) % (n_pad - n))
  src2 = jnp.concatenate([src, pad_idx]).reshape(NW, per_w)
  dst_flat = jnp.concatenate([dst, pad_idx])
  dst2 = dst_flat.reshape(NW * n_chunks, CH)
  dst_deg = dst_flat.reshape(NW, per_w // DCH, DCH)
  x_pad = jnp.concatenate([x, jnp.zeros((n_pad - n, d), jnp.float32)])

  d0, d1 = _make_deg_kernel(n_pad, e_pad)(
      dst_deg, jnp.zeros((n_pad,), jnp.float32))
  d0c = d0.reshape(n_pad, 1)
  d1c = d1.reshape(n_pad, 1)

  grid = (n_pad // BLK,)
  row_spec = pl.BlockSpec((BLK, d), lambda i: (i, 0))
  col_spec = pl.BlockSpec((BLK, 1), lambda i: (i, 0))
  vec_spec = pl.BlockSpec((1, d), lambda i: (0, 0))

  h2 = pl.pallas_call(
      _prescale_body,
      grid=grid,
      in_specs=[row_spec, pl.BlockSpec((d, d), lambda i: (0, 0)),
                col_spec, col_spec],
      out_specs=row_spec,
      out_shape=jax.ShapeDtypeStruct((n_pad, d), jnp.float32),
  )(x_pad, W, d0c, d1c)

  p0, p1 = _make_edge_kernel(n_pad, e_pad, d)(h2, src2, dst2)

  out_pad = pl.pallas_call(
      _finalize_body,
      grid=grid,
      in_specs=[row_spec, row_spec, row_spec, row_spec, col_spec, col_spec,
                vec_spec, vec_spec, vec_spec],
      out_specs=row_spec,
      out_shape=jax.ShapeDtypeStruct((n_pad, d), jnp.float32),
  )(p0, p1, h2, x_pad, d0c, d1c,
    b.reshape(1, d), gamma.reshape(1, d), beta.reshape(1, d))

  return out_pad[:n]


# speedup vs baseline: 1.1165x; 1.0009x over previous
import functools

import jax
import jax.numpy as jnp
from jax import lax
from jax.experimental import pallas as pl
from jax.experimental.pallas import tpu as pltpu
from jax.experimental.pallas import tpu_sc as plsc

NC = 2
NS = 16
NW = NC * NS
DCH = 128
CH = 64
NBUF = 4
BLK = 1280


def _sc_mesh():
  return plsc.VectorSubcoreMesh(
      core_axis_name="c", subcore_axis_name="s",
      num_cores=NC, num_subcores=NS)


def _make_deg_kernel(n_pad, e_pad):
  per_w = e_pad // NW
  n_chunks = per_w // DCH
  rows_per_s = n_pad // NS

  @functools.partial(
      pl.kernel,
      out_type=(jax.ShapeDtypeStruct((n_pad,), jnp.float32),
                jax.ShapeDtypeStruct((n_pad,), jnp.float32)),
      mesh=_sc_mesh(),
      scratch_types=[
          pltpu.VMEM_SHARED((n_pad,), jnp.float32),
          pltpu.VMEM((n_chunks, DCH), jnp.int32),
          pltpu.VMEM((DCH,), jnp.float32),
          pltpu.SemaphoreType.DMA,
      ],
  )
  def deg_kernel(dst_hbm, zeros_hbm, d0_hbm, d1_hbm,
                 deg_sh, dsts_v, ones_v, sem):
    c = lax.axis_index("c")
    sax = lax.axis_index("s")
    wid = c * NS + sax
    for j in range(DCH // 16):
      ones_v[pl.ds(j * 16, 16)] = jnp.ones((16,), jnp.float32)
    row0 = sax * rows_per_s
    zcp = pltpu.async_copy(zeros_hbm.at[pl.ds(row0, rows_per_s)],
                           deg_sh.at[pl.ds(row0, rows_per_s)], sem)
    icp = pltpu.async_copy(dst_hbm.at[wid], dsts_v, sem)
    zcp.wait()
    icp.wait()
    plsc.subcore_barrier()

    def fire(i, carry):
      pltpu.async_copy(ones_v, deg_sh.at[dsts_v.at[i]], sem, add=True)
      return carry

    lax.fori_loop(0, n_chunks, fire, 0)

    def drain(i, carry):
      pltpu.make_async_copy(zeros_hbm.at[pl.ds(0, DCH)], ones_v, sem).wait()
      return carry

    lax.fori_loop(0, n_chunks, drain, 0)
    plsc.subcore_barrier()

    @pl.when(c == 0)
    def _():
      pltpu.sync_copy(deg_sh.at[pl.ds(row0, rows_per_s)],
                      d0_hbm.at[pl.ds(row0, rows_per_s)])

    @pl.when(c == 1)
    def _():
      pltpu.sync_copy(deg_sh.at[pl.ds(row0, rows_per_s)],
                      d1_hbm.at[pl.ds(row0, rows_per_s)])

  return deg_kernel


def _make_edge_kernel(n_pad, e_pad, d):
  per_w = e_pad // NW
  n_chunks = per_w // CH
  n_rounds = n_chunks // NBUF
  rows_per_s = n_pad // NS

  @functools.partial(
      pl.kernel,
      out_type=(jax.ShapeDtypeStruct((n_pad, d), jnp.float32),
                jax.ShapeDtypeStruct((n_pad, d), jnp.float32)),
      mesh=_sc_mesh(),
      scratch_types=[
          pltpu.VMEM_SHARED((n_pad, d), jnp.float32),
          pltpu.VMEM((per_w,), jnp.int32),
      ] + [pltpu.VMEM((CH,), jnp.int32)] * NBUF
        + [pltpu.VMEM((CH, d), jnp.float32)] * NBUF
        + [pltpu.SemaphoreType.DMA] * (3 * NBUF),
  )
  def edge_kernel(h2_hbm, src_hbm, dst_hbm, p0_hbm, p1_hbm,
                  acc_sh, srcs_v, *bufs_and_sems):
    dstb = bufs_and_sems[:NBUF]
    rows = bufs_and_sems[NBUF:2 * NBUF]
    gsem = bufs_and_sems[2 * NBUF:3 * NBUF]
    ssem = bufs_and_sems[3 * NBUF:4 * NBUF]
    dsem = bufs_and_sems[4 * NBUF:]
    c = lax.axis_index("c")
    sax = lax.axis_index("s")
    wid = c * NS + sax
    row0 = sax * rows_per_s
    init_cp = pltpu.async_copy(h2_hbm.at[pl.ds(row0, rows_per_s)],
                               acc_sh.at[pl.ds(row0, rows_per_s)], gsem[0])
    src_cp = pltpu.async_copy(src_hbm.at[wid], srcs_v, gsem[1])
    init_cp.wait()
    src_cp.wait()
    plsc.subcore_barrier()

    dst_base = wid * n_chunks
    for b in range(NBUF):
      pltpu.async_copy(dst_hbm.at[dst_base + b], dstb[b], dsem[b])
      pltpu.async_copy(h2_hbm.at[srcs_v.at[pl.ds(b * CH, CH)]],
                       rows[b], gsem[b])

    def round_body(j, carry):
      for b in range(NBUF):
        pltpu.make_async_copy(dst_hbm.at[0], dstb[b], dsem[b]).wait()
        pltpu.make_async_copy(h2_hbm.at[pl.ds(0, CH)], rows[b],
                              gsem[b]).wait()
        pltpu.async_copy(rows[b], acc_sh.at[dstb[b]], ssem[b], add=True)
      for b in range(NBUF):
        nxt = NBUF * (j + 1) + b
        pltpu.make_async_copy(h2_hbm.at[pl.ds(0, CH)], rows[b],
                              ssem[b]).wait()

        @pl.when(nxt < n_chunks)
        def _():
          pltpu.async_copy(dst_hbm.at[dst_base + nxt], dstb[b], dsem[b])
          pltpu.async_copy(h2_hbm.at[srcs_v.at[pl.ds(nxt * CH, CH)]],
                           rows[b], gsem[b])

      return carry

    lax.fori_loop(0, n_rounds, round_body, 0)
    plsc.subcore_barrier()

    @pl.when(c == 0)
    def _():
      pltpu.sync_copy(acc_sh.at[pl.ds(row0, rows_per_s)],
                      p0_hbm.at[pl.ds(row0, rows_per_s)])

    @pl.when(c == 1)
    def _():
      pltpu.sync_copy(acc_sh.at[pl.ds(row0, rows_per_s)],
                      p1_hbm.at[pl.ds(row0, rows_per_s)])

  return edge_kernel


def _prescale_body(x_ref, w_ref, d0_ref, d1_ref, h2_ref):
  dis = lax.rsqrt(1.0 + d0_ref[...] + d1_ref[...])
  h2_ref[...] = jnp.dot(x_ref[...] * dis, w_ref[...],
                        preferred_element_type=jnp.float32)


def _finalize_body(p0_ref, p1_ref, h2_ref, x_ref, d0_ref, d1_ref,
                   b_ref, g_ref, be_ref, o_ref):
  dis = lax.rsqrt(1.0 + d0_ref[...] + d1_ref[...])
  acc = p0_ref[...] + p1_ref[...] - h2_ref[...]
  out = acc * dis + b_ref[...]
  mu = jnp.mean(out, axis=-1, keepdims=True)
  var = jnp.mean((out - mu) ** 2, axis=-1, keepdims=True)
  ln = (out - mu) * lax.rsqrt(var + 1e-5) * g_ref[...] + be_ref[...]
  o_ref[...] = jnp.maximum(ln + x_ref[...], 0.0)


def kernel(x, edge_index, W, b, gamma, beta):
  n, d = x.shape
  e = edge_index.shape[1]
  n_pad = ((n + BLK) // BLK) * BLK
  egrain = NW * max(CH * NBUF, DCH)
  e_pad = ((e + egrain - 1) // egrain) * egrain
  per_w = e_pad // NW
  n_chunks = per_w // CH

  src = edge_index[0].astype(jnp.int32)
  dst = edge_index[1].astype(jnp.int32)
  pad_idx = n + (jnp.arange(e_pad - e, dtype=jnp.int32) % (n_pad - n))
  src2 = jnp.concatenate([src, pad_idx]).reshape(NW, per_w)
  dst_flat = jnp.concatenate([dst, pad_idx])
  dst2 = dst_flat.reshape(NW * n_chunks, CH)
  dst_deg = dst_flat.reshape(NW, per_w // DCH, DCH)
  x_pad = jnp.concatenate([x, jnp.zeros((n_pad - n, d), jnp.float32)])

  d0, d1 = _make_deg_kernel(n_pad, e_pad)(
      dst_deg, jnp.zeros((n_pad,), jnp.float32))
  d0c = d0.reshape(n_pad, 1)
  d1c = d1.reshape(n_pad, 1)

  grid = (n_pad // BLK,)
  row_spec = pl.BlockSpec((BLK, d), lambda i: (i, 0))
  col_spec = pl.BlockSpec((BLK, 1), lambda i: (i, 0))
  vec_spec = pl.BlockSpec((1, d), lambda i: (0, 0))

  h2 = pl.pallas_call(
      _prescale_body,
      grid=grid,
      in_specs=[row_spec, pl.BlockSpec((d, d), lambda i: (0, 0)),
                col_spec, col_spec],
      out_specs=row_spec,
      out_shape=jax.ShapeDtypeStruct((n_pad, d), jnp.float32),
  )(x_pad, W, d0c, d1c)

  p0, p1 = _make_edge_kernel(n_pad, e_pad, d)(h2, src2, dst2)

  out_pad = pl.pallas_call(
      _finalize_body,
      grid=grid,
      in_specs=[row_spec, row_spec, row_spec, row_spec, col_spec, col_spec,
                vec_spec, vec_spec, vec_spec],
      out_specs=row_spec,
      out_shape=jax.ShapeDtypeStruct((n_pad, d), jnp.float32),
  )(p0, p1, h2, x_pad, d0c, d1c,
    b.reshape(1, d), gamma.reshape(1, d), beta.reshape(1, d))

  return out_pad[:n]
